# baseline (speedup 1.0000x reference)
import functools

import jax
import jax.numpy as jnp
from jax import lax
from jax.experimental import pallas as pl
from jax.experimental.pallas import tpu as pltpu

_EPS = 1e-5
_S = 112
_N = _S * _S
_TOPK = 200
_CIN = 256
_RB = 8
_NBLK = _S // _RB


def _colshift(a, k, fill):
    c = a.shape[1]
    col = lax.broadcasted_iota(jnp.int32, (a.shape[0], 1), 0) % _S
    pad = jnp.full((1, c), fill, a.dtype)
    if k == -1:
        sh = jnp.concatenate([pad, a[:-1]], axis=0)
        return jnp.where(col != 0, sh, fill)
    if k == 1:
        sh = jnp.concatenate([a[1:], pad], axis=0)
        return jnp.where(col != _S - 1, sh, fill)
    return a


def _colmask(a, kc):
    if kc == 1:
        return a
    col = lax.broadcasted_iota(jnp.int32, (a.shape[0], 1), 0) % _S
    bad = 0 if kc == 0 else _S - 1
    return jnp.where(col != bad, a, 0.0)


def _mm(a, b):
    return jnp.dot(a, b, preferred_element_type=jnp.float32,
                   precision=lax.Precision.HIGHEST)


def _conv1_from_ref(x_ref, r0, w_ref, bias, nout):
    h = jnp.zeros((nout * _S, _CIN), jnp.float32) + bias
    for kr in range(3):
        base = (r0 + 1 + kr) * _S - 8
        sup = x_ref[0, pl.ds(base, nout * _S + 16), :]
        for kc in range(3):
            seg = sup[8 + kc - 1:8 + kc - 1 + nout * _S]
            h = h + _colmask(_mm(seg, w_ref[kr * 3 + kc]), kc)
    return jnp.maximum(h, 0.0)


def _conv2_from_val(strip, w_ref, bias, nout):
    zs = jnp.zeros((_S, _CIN), jnp.float32)
    he = jnp.concatenate([zs, strip, zs], axis=0)
    h = jnp.zeros((nout * _S, _CIN), jnp.float32) + bias
    for kc in range(3):
        a = jnp.zeros((nout * _S, _CIN), jnp.float32)
        for kr in range(3):
            start = (kr + 1) * _S + kc - 1
            seg = he[start:start + nout * _S]
            a = a + _mm(seg, w_ref[kr * 3 + kc])
        h = h + _colmask(a, kc)
    return jnp.maximum(h, 0.0)


def _rowmask(nrows, r0_first, fill, a):
    g = lax.broadcasted_iota(jnp.int32, (nrows * _S, 1), 0) // _S + r0_first
    return jnp.where((g >= 0) & (g < _S), a, fill)


def _conv_kernel(x_ref, w0_ref, c0_ref, w1_ref, c1_ref, w2_ref, b2_ref,
                 act_ref, out_ref):
    neg = jnp.float32(-jnp.inf)
    blk = pl.program_id(1)
    r0 = blk * _RB

    h1 = _conv1_from_ref(x_ref, r0, w0_ref, c0_ref[...], _RB + 4)
    h1 = _rowmask(_RB + 4, r0 - 2, jnp.float32(0.0), h1)
    h2 = _conv2_from_val(h1, w1_ref, c1_ref[...], _RB + 2)
    out16 = jnp.dot(h2, w2_ref[...], preferred_element_type=jnp.float32,
                    precision=lax.Precision.HIGHEST) + b2_ref[...]
    ctr = out16[_S:(_RB + 1) * _S]

    act = act_ref[0, pl.ds(r0 * _S, (_RB + 2) * _S), :]
    act = _rowmask(_RB + 2, r0 - 1, neg, act)
    actc = act[_S:(_RB + 1) * _S]
    m = actc
    for kr in range(3):
        seg = act[kr * _S:(kr + _RB) * _S]
        for kc in (-1, 0, 1):
            if kr == 1 and kc == 0:
                continue
            m = jnp.maximum(m, _colshift(seg, kc, neg))
    msk = actc * (actc == m).astype(jnp.float32)

    out_ref[0] = jnp.concatenate(
        [ctr, jax.nn.sigmoid(ctr[:, 12:16]), msk], axis=1)


def _dense_stage(xpad, w0f, c0, w1f, c1, w2f, b2, act3):
    B = xpad.shape[0]
    return pl.pallas_call(
        _conv_kernel,
        grid=(B, _NBLK),
        in_specs=[
            pl.BlockSpec((1, (_S + 8) * _S, _CIN), lambda b, j: (b, 0, 0)),
            pl.BlockSpec((9, _CIN, _CIN), lambda b, j: (0, 0, 0)),
            pl.BlockSpec((1, _CIN), lambda b, j: (0, 0)),
            pl.BlockSpec((9, _CIN, _CIN), lambda b, j: (0, 0, 0)),
            pl.BlockSpec((1, _CIN), lambda b, j: (0, 0)),
            pl.BlockSpec((_CIN, 16), lambda b, j: (0, 0)),
            pl.BlockSpec((1, 16), lambda b, j: (0, 0)),
            pl.BlockSpec((1, (_S + 2) * _S, 3), lambda b, j: (b, 0, 0)),
        ],
        out_specs=pl.BlockSpec((1, _RB * _S, 23), lambda b, j: (b, j, 0)),
        out_shape=jax.ShapeDtypeStruct((B, _N, 23), jnp.float32),
    )(xpad, w0f, c0, w1f, c1, w2f, b2, act3)


def kernel(x, w0, g0, b0, m0, v0, w1, g1, b1, m1, v1, w2, b2):
    B = x.shape[0]
    s0 = g0 / jnp.sqrt(v0 + _EPS)
    s1 = g1 / jnp.sqrt(v1 + _EPS)
    w0f = (w0 * s0[None, None, None, :]).reshape(9, _CIN, _CIN)
    w1f = (w1 * s1[None, None, None, :]).reshape(9, _CIN, _CIN)
    c0 = (b0 - m0 * s0).reshape(1, _CIN)
    c1 = (b1 - m1 * s1).reshape(1, _CIN)
    w2f = w2.reshape(_CIN, 16)
    b2f = b2.reshape(1, 16)

    def _conv(inp, w):
        return lax.conv_general_dilated(
            inp, w, (1, 1), "SAME", rhs_dilation=(1, 1),
            dimension_numbers=("NHWC", "HWIO", "NHWC"))

    def _bn(h, g, bb, m, v):
        return (h - m) / jnp.sqrt(v + _EPS) * g + bb

    h1x = jax.nn.relu(_bn(_conv(x, w0), g0, b0, m0, v0))
    h2x = jax.nn.relu(_bn(_conv(h1x, w1), g1, b1, m1, v1))
    o16x = _conv(h2x, w2) + b2
    act3 = jax.nn.sigmoid(
        jnp.concatenate([o16x[:, :, :, 7:8], o16x[:, :, :, 0:1],
                         o16x[:, :, :, 2:3]], axis=-1)).reshape(B, _N, 3)
    zp1 = jnp.zeros((B, _S, 3), jnp.float32)
    act3p = jnp.concatenate([zp1, act3, zp1], axis=1)

    xf = x.reshape(B, _N, _CIN)
    zpad = jnp.zeros((B, 4 * _S, _CIN), jnp.float32)
    xpad = jnp.concatenate([zpad, xf, zpad], axis=1)
    comb = _dense_stage(xpad, w0f, c0, w1f, c1, w2f, b2f, act3p)

    out_map = comb[:, :, :16].reshape(B, _S, _S, 16)
    sig4 = comb[:, :, 16:20].reshape(B, _S, _S, 4)
    mskf = comb[:, :, 20:23]

    center_map = out_map[:, :, :, 0:1]
    line_map = out_map[:, :, :, 1:2]
    corner_map = out_map[:, :, :, 2:3]
    disp_map = out_map[:, :, :, 3:7]
    org_center_map = out_map[:, :, :, 7:8]
    org_disp_map = out_map[:, :, :, 8:12]
    org_dist_map = sig4[:, :, :, 0:1]
    org_deg_map = sig4[:, :, :, 1:2]
    split_dist_map = sig4[:, :, :, 2:3]
    split_deg_map = sig4[:, :, :, 3:4]

    flat = mskf.transpose(0, 2, 1)
    scores, idx = lax.top_k(flat.reshape(B * 3, _N), _TOPK)
    scores = scores.reshape(B, 3, _TOPK)
    idx = idx.reshape(B, 3, _TOPK)
    pts = jnp.stack([idx // _S, idx % _S], axis=-1)

    org_center_pts, org_center_scores = pts[:, 0], scores[:, 0]
    center_pts, center_scores = pts[:, 1], scores[:, 1]
    corner_pts, corner_scores = pts[:, 2], scores[:, 2]

    return (center_map, disp_map, center_pts, center_scores, disp_map,
            line_map, corner_map, corner_pts, corner_scores,
            org_center_map, org_disp_map, org_center_pts, org_center_scores,
            org_dist_map, org_deg_map, split_dist_map, split_deg_map)


# speedup vs baseline: 1.0748x; 1.0748x over previous
import functools

import jax
import jax.numpy as jnp
from jax import lax
from jax.experimental import pallas as pl
from jax.experimental.pallas import tpu as pltpu
from jax.experimental.pallas import tpu_sc as plsc

_EPS = 1e-5
_S = 112
_N = _S * _S
_TOPK = 200
_CIN = 256
_RB = 8
_NBLK = _S // _RB


def _colshift(a, k, fill):
    c = a.shape[1]
    col = lax.broadcasted_iota(jnp.int32, (a.shape[0], 1), 0) % _S
    pad = jnp.full((1, c), fill, a.dtype)
    if k == -1:
        sh = jnp.concatenate([pad, a[:-1]], axis=0)
        return jnp.where(col != 0, sh, fill)
    if k == 1:
        sh = jnp.concatenate([a[1:], pad], axis=0)
        return jnp.where(col != _S - 1, sh, fill)
    return a


def _colmask(a, kc):
    if kc == 1:
        return a
    col = lax.broadcasted_iota(jnp.int32, (a.shape[0], 1), 0) % _S
    bad = 0 if kc == 0 else _S - 1
    return jnp.where(col != bad, a, 0.0)


def _mm(a, b):
    return jnp.dot(a, b, preferred_element_type=jnp.float32,
                   precision=lax.Precision.HIGHEST)


def _conv1_from_ref(x_ref, r0, w_ref, bias, nout):
    h = jnp.zeros((nout * _S, _CIN), jnp.float32) + bias
    for kr in range(3):
        base = (r0 + 1 + kr) * _S - 8
        sup = x_ref[0, pl.ds(base, nout * _S + 16), :]
        for kc in range(3):
            seg = sup[8 + kc - 1:8 + kc - 1 + nout * _S]
            h = h + _colmask(_mm(seg, w_ref[kr * 3 + kc]), kc)
    return jnp.maximum(h, 0.0)


def _conv2_from_val(strip, w_ref, bias, nout):
    zs = jnp.zeros((_S, _CIN), jnp.float32)
    he = jnp.concatenate([zs, strip, zs], axis=0)
    h = jnp.zeros((nout * _S, _CIN), jnp.float32) + bias
    for kc in range(3):
        a = jnp.zeros((nout * _S, _CIN), jnp.float32)
        for kr in range(3):
            start = (kr + 1) * _S + kc - 1
            seg = he[start:start + nout * _S]
            a = a + _mm(seg, w_ref[kr * 3 + kc])
        h = h + _colmask(a, kc)
    return jnp.maximum(h, 0.0)


def _rowmask(nrows, r0_first, fill, a):
    g = lax.broadcasted_iota(jnp.int32, (nrows * _S, 1), 0) // _S + r0_first
    return jnp.where((g >= 0) & (g < _S), a, fill)


def _conv_kernel(x_ref, w0_ref, c0_ref, w1_ref, c1_ref, w2_ref, b2_ref,
                 act_ref, out_ref):
    neg = jnp.float32(-jnp.inf)
    blk = pl.program_id(1)
    r0 = blk * _RB

    h1 = _conv1_from_ref(x_ref, r0, w0_ref, c0_ref[...], _RB + 4)
    h1 = _rowmask(_RB + 4, r0 - 2, jnp.float32(0.0), h1)
    h2 = _conv2_from_val(h1, w1_ref, c1_ref[...], _RB + 2)
    out16 = jnp.dot(h2, w2_ref[...], preferred_element_type=jnp.float32,
                    precision=lax.Precision.HIGHEST) + b2_ref[...]
    ctr = out16[_S:(_RB + 1) * _S]

    act = act_ref[0, pl.ds(r0 * _S, (_RB + 2) * _S), :]
    act = _rowmask(_RB + 2, r0 - 1, neg, act)
    actc = act[_S:(_RB + 1) * _S]
    m = actc
    for kr in range(3):
        seg = act[kr * _S:(kr + _RB) * _S]
        for kc in (-1, 0, 1):
            if kr == 1 and kc == 0:
                continue
            m = jnp.maximum(m, _colshift(seg, kc, neg))
    msk = actc * (actc == m).astype(jnp.float32)

    out_ref[0] = jnp.concatenate(
        [ctr, jax.nn.sigmoid(ctr[:, 12:16]), msk], axis=1)


def _dense_stage(xpad, w0f, c0, w1f, c1, w2f, b2, act3):
    B = xpad.shape[0]
    return pl.pallas_call(
        _conv_kernel,
        grid=(B, _NBLK),
        in_specs=[
            pl.BlockSpec((1, (_S + 8) * _S, _CIN), lambda b, j: (b, 0, 0)),
            pl.BlockSpec((9, _CIN, _CIN), lambda b, j: (0, 0, 0)),
            pl.BlockSpec((1, _CIN), lambda b, j: (0, 0)),
            pl.BlockSpec((9, _CIN, _CIN), lambda b, j: (0, 0, 0)),
            pl.BlockSpec((1, _CIN), lambda b, j: (0, 0)),
            pl.BlockSpec((_CIN, 16), lambda b, j: (0, 0)),
            pl.BlockSpec((1, 16), lambda b, j: (0, 0)),
            pl.BlockSpec((1, (_S + 2) * _S, 3), lambda b, j: (b, 0, 0)),
        ],
        out_specs=pl.BlockSpec((1, _RB * _S, 23), lambda b, j: (b, j, 0)),
        out_shape=jax.ShapeDtypeStruct((B, _N, 23), jnp.float32),
    )(xpad, w0f, c0, w1f, c1, w2f, b2, act3)


_CAP = 1024
_NCH = _N // 16
_OPAD = 208


def _topk_kernel(in_hbm, bits_hbm, outv_hbm, outi_hbm, vals, bitsv, hist,
                 cv, ci_, ov, oi):
    wid = lax.axis_index("s") * 2 + lax.axis_index("c")

    @pl.when(wid < 12)
    def _():
        pltpu.sync_copy(in_hbm.at[wid], vals)
        pltpu.sync_copy(bits_hbm.at[wid], bitsv)
        lane = lax.iota(jnp.int32, 16)
        ones = jnp.ones((16,), jnp.int32)
        zi16 = jnp.zeros((16,), jnp.int32)

        prefix = jnp.int32(0)
        rem = jnp.int32(_TOPK)
        tm = jnp.int32(_N)
        for p in range(4):
            sh = 24 - 8 * p

            def zbody(k, _):
                hist[pl.ds(k * 16, 16)] = zi16
                return 0

            lax.fori_loop(0, 256, zbody, 0)

            def hbody(i, pfx, sh=sh, p=p):
                b = bitsv[pl.ds(i * 16, 16)]
                digit = jnp.bitwise_and(lax.shift_right_logical(b, sh), 255)
                if p == 0:
                    mv = jnp.ones((16,), jnp.bool_)
                else:
                    mv = lax.shift_right_logical(b, sh + 8) == \
                        lax.shift_right_logical(pfx, sh + 8)
                plsc.addupdate_scatter(hist, [lane * 256 + digit], ones,
                                       mask=mv)
                return pfx

            prefix = lax.fori_loop(0, _NCH, hbody, prefix)

            target = tm - rem

            def fbody(k, c):
                found, dd, cincl, cnt, base = c
                acc = hist[pl.ds(k * 16, 16)]
                for l in range(1, 16):
                    acc = acc + hist[pl.ds(l * 256 + k * 16, 16)]
                cum = plsc.cumsum(acc) + base
                mm = cum > target
                nset = jnp.sum(mm.astype(jnp.int32))
                take = jnp.logical_and(found == 0, nset > 0)
                ffs = plsc.all_reduce_ffs(mm)
                sel = lane == ffs
                d_new = k * 16 + jnp.sum(jnp.where(sel, lane, 0))
                ci_new = jnp.sum(jnp.where(sel, cum, 0))
                cnt_new = jnp.sum(jnp.where(sel, acc, 0))
                return (jnp.where(take, 1, found),
                        jnp.where(take, d_new, dd),
                        jnp.where(take, ci_new, cincl),
                        jnp.where(take, cnt_new, cnt),
                        base + jnp.sum(acc))

            _, dbin, cincl, cnt, _ = lax.fori_loop(
                0, 16, fbody, (jnp.int32(0), jnp.int32(0), jnp.int32(0),
                               jnp.int32(0), jnp.int32(0)))
            rem = rem - (tm - cincl)
            tm = cnt
            prefix = jnp.bitwise_or(prefix, lax.shift_left(dbin, sh))

        def pbody(i, _):
            cv[pl.ds(i * 16, 16)] = jnp.full((16,), -1.0, jnp.float32)
            ci_[pl.ds(i * 16, 16)] = zi16
            return 0

        lax.fori_loop(0, (_CAP + 16) // 16, pbody, 0)

        def cbody(i, n):
            v = vals[pl.ds(i * 16, 16)]
            b = bitsv[pl.ds(i * 16, 16)]
            mm = b >= prefix
            plsc.store_compressed(cv.at[pl.ds(n, 16)], v, mask=mm)
            plsc.store_compressed(ci_.at[pl.ds(n, 16)], lane + i * 16, mask=mm)
            return jnp.minimum(n + jnp.sum(mm.astype(jnp.int32)),
                               jnp.int32(_CAP))

        n = lax.fori_loop(0, _NCH, cbody, jnp.int32(0))

        def sbody(k, _):
            gl = lane + k * 16
            v = cv[pl.ds(k * 16, 16)]
            ix = ci_[pl.ds(k * 16, 16)]
            cv[pl.ds(k * 16, 16)] = jnp.where(gl >= n, -1.0, v)
            ci_[pl.ds(k * 16, 16)] = jnp.where(gl >= n, jnp.int32(2**31 - 1),
                                               ix)
            return 0

        lax.fori_loop(0, (_CAP + 16) // 16, sbody, 0)

        def obody(z, _):
            ov[pl.ds(z * 16, 16)] = jnp.zeros((16,), jnp.float32)
            oi[pl.ds(z * 16, 16)] = zi16
            return 0

        lax.fori_loop(0, _OPAD // 16, obody, 0)
        nch = (n + 15) // 16

        def outer(cib, _):
            vi = cv[pl.ds(cib * 16, 16)]
            ii = ci_[pl.ds(cib * 16, 16)]

            def inner(cjb, rank):
                vj = cv[pl.ds(cjb * 16, 16)]
                ij = ci_[pl.ds(cjb * 16, 16)]
                for l in range(16):
                    s = vj[l]
                    si = ij[l]
                    beat = jnp.logical_or(
                        s > vi, jnp.logical_and(s == vi, si < ii))
                    rank = rank + jnp.where(beat, 1, 0)
                return rank

            rank = lax.fori_loop(0, nch, inner, jnp.zeros((16,), jnp.int32))
            okm = jnp.logical_and(rank < _TOPK, lane + cib * 16 < n)
            plsc.store_scatter(ov, [rank], vi, mask=okm)
            plsc.store_scatter(oi, [rank], ii, mask=okm)
            return 0

        lax.fori_loop(0, nch, outer, 0)
        pltpu.sync_copy(ov, outv_hbm.at[wid])
        pltpu.sync_copy(oi, outi_hbm.at[wid])


def _topk_sc(masked12):
    import functools
    mesh = plsc.VectorSubcoreMesh(core_axis_name="c", subcore_axis_name="s")
    f = functools.partial(
        pl.kernel, mesh=mesh,
        compiler_params=pltpu.CompilerParams(needs_layout_passes=False),
        out_type=[jax.ShapeDtypeStruct((12, _OPAD), jnp.float32),
                  jax.ShapeDtypeStruct((12, _OPAD), jnp.int32)],
        scratch_types=[pltpu.VMEM((_N,), jnp.float32),
                       pltpu.VMEM((_N,), jnp.int32),
                       pltpu.VMEM((4096,), jnp.int32),
                       pltpu.VMEM((_CAP + 16,), jnp.float32),
                       pltpu.VMEM((_CAP + 16,), jnp.int32),
                       pltpu.VMEM((_OPAD,), jnp.float32),
                       pltpu.VMEM((_OPAD,), jnp.int32)],
    )(_topk_kernel)
    return f(masked12, lax.bitcast_convert_type(masked12, jnp.int32))


def kernel(x, w0, g0, b0, m0, v0, w1, g1, b1, m1, v1, w2, b2):
    B = x.shape[0]
    s0 = g0 / jnp.sqrt(v0 + _EPS)
    s1 = g1 / jnp.sqrt(v1 + _EPS)
    w0f = (w0 * s0[None, None, None, :]).reshape(9, _CIN, _CIN)
    w1f = (w1 * s1[None, None, None, :]).reshape(9, _CIN, _CIN)
    c0 = (b0 - m0 * s0).reshape(1, _CIN)
    c1 = (b1 - m1 * s1).reshape(1, _CIN)
    w2f = w2.reshape(_CIN, 16)
    b2f = b2.reshape(1, 16)

    def _conv(inp, w):
        return lax.conv_general_dilated(
            inp, w, (1, 1), "SAME", rhs_dilation=(1, 1),
            dimension_numbers=("NHWC", "HWIO", "NHWC"))

    def _bn(h, g, bb, m, v):
        return (h - m) / jnp.sqrt(v + _EPS) * g + bb

    h1x = jax.nn.relu(_bn(_conv(x, w0), g0, b0, m0, v0))
    h2x = jax.nn.relu(_bn(_conv(h1x, w1), g1, b1, m1, v1))
    o16x = _conv(h2x, w2) + b2
    act3 = jax.nn.sigmoid(
        jnp.concatenate([o16x[:, :, :, 7:8], o16x[:, :, :, 0:1],
                         o16x[:, :, :, 2:3]], axis=-1)).reshape(B, _N, 3)
    zp1 = jnp.zeros((B, _S, 3), jnp.float32)
    act3p = jnp.concatenate([zp1, act3, zp1], axis=1)

    xf = x.reshape(B, _N, _CIN)
    zpad = jnp.zeros((B, 4 * _S, _CIN), jnp.float32)
    xpad = jnp.concatenate([zpad, xf, zpad], axis=1)
    comb = _dense_stage(xpad, w0f, c0, w1f, c1, w2f, b2f, act3p)

    out_map = comb[:, :, :16].reshape(B, _S, _S, 16)
    sig4 = comb[:, :, 16:20].reshape(B, _S, _S, 4)
    mskf = comb[:, :, 20:23]

    center_map = out_map[:, :, :, 0:1]
    line_map = out_map[:, :, :, 1:2]
    corner_map = out_map[:, :, :, 2:3]
    disp_map = out_map[:, :, :, 3:7]
    org_center_map = out_map[:, :, :, 7:8]
    org_disp_map = out_map[:, :, :, 8:12]
    org_dist_map = sig4[:, :, :, 0:1]
    org_deg_map = sig4[:, :, :, 1:2]
    split_dist_map = sig4[:, :, :, 2:3]
    split_deg_map = sig4[:, :, :, 3:4]

    flat = mskf.transpose(0, 2, 1).reshape(B * 3, _N)
    scores_p, idx_p = _topk_sc(flat)
    scores = scores_p[:, :_TOPK].reshape(B, 3, _TOPK)
    idx = idx_p[:, :_TOPK].reshape(B, 3, _TOPK)
    pts = jnp.stack([idx // _S, idx % _S], axis=-1)

    org_center_pts, org_center_scores = pts[:, 0], scores[:, 0]
    center_pts, center_scores = pts[:, 1], scores[:, 1]
    corner_pts, corner_scores = pts[:, 2], scores[:, 2]

    return (center_map, disp_map, center_pts, center_scores, disp_map,
            line_map, corner_map, corner_pts, corner_scores,
            org_center_map, org_disp_map, org_center_pts, org_center_scores,
            org_dist_map, org_deg_map, split_dist_map, split_deg_map)
